# Initial kernel scaffold; baseline (speedup 1.0000x reference)
import functools

import jax
import jax.numpy as jnp
from jax import lax
from jax.experimental import pallas as pl
from jax.experimental.pallas import tpu as pltpu
from jax.experimental.pallas import tpu_sc as plsc

VOCAB = 100000
EMBED = 128
BATCH = 16384

_INFO = plsc.get_sparse_core_info()
_NC, _NS, _L = _INFO.num_cores, _INFO.num_subcores, _INFO.num_lanes
_NW = _NC * _NS
_BPW = BATCH // _NW
_CHUNK = 128
_NCHUNK = _BPW // _CHUNK
_GROUPS = _CHUNK // 16


def _sc_body(word_hbm, ctx_hbm, wt_hbm, ct_hbm, scal_hbm, out_hbm,
             idx_w, idx_c, wrows, crows, tbuf, res, scal_v, sem_w, sem_c):
    wid = lax.axis_index("s") * _NC + lax.axis_index("c")
    base = wid * _BPW

    for c in range(_NCHUNK):
        pltpu.sync_copy(word_hbm.at[pl.ds(base + c * _CHUNK, _CHUNK)],
                        idx_w.at[c])
        pltpu.sync_copy(ctx_hbm.at[pl.ds(base + c * _CHUNK, _CHUNK)],
                        idx_c.at[c])
    pltpu.sync_copy(scal_hbm, scal_v)
    wvec = scal_v[0, :]
    bvec = scal_v[1, :]
    iota16 = lax.iota(jnp.int32, _L)

    for c in range(_NCHUNK):
        cw = pltpu.async_copy(wt_hbm.at[idx_w.at[c]], wrows, sem_w)
        cc = pltpu.async_copy(ct_hbm.at[idx_c.at[c]], crows, sem_c)
        cw.wait()
        cc.wait()

        def group_body(g, _, c=c):
            row0 = g * 16
            for r in range(16):
                row = row0 + r
                p = wrows[row, pl.ds(0, 16)] * crows[row, pl.ds(0, 16)]
                for k in range(1, EMBED // 16):
                    p = p + (wrows[row, pl.ds(k * 16, 16)] *
                             crows[row, pl.ds(k * 16, 16)])
                tbuf[r, :] = p
            acc = plsc.load_gather(tbuf, [iota16, jnp.zeros((16,), jnp.int32)])
            for j in range(1, 16):
                acc = acc + plsc.load_gather(
                    tbuf, [iota16, jnp.full((16,), j, jnp.int32)])
            z = acc * wvec + bvec
            res[pl.ds(c * _CHUNK + row0, 16)] = 1.0 / (1.0 + jnp.exp(-z))
            return 0

        lax.fori_loop(0, _GROUPS, group_body, 0)

    pltpu.sync_copy(res, out_hbm.at[pl.ds(base, _BPW)])


def kernel(word, context, word_table, ctx_table, dense_w, dense_b):
    word_i = word.reshape(-1).astype(jnp.int32)
    ctx_i = context.reshape(-1).astype(jnp.int32)
    scal = jnp.stack([
        jnp.broadcast_to(dense_w.reshape(()), (_L,)),
        jnp.broadcast_to(dense_b.reshape(()), (_L,)),
    ]).astype(jnp.float32)

    mesh = plsc.VectorSubcoreMesh(core_axis_name="c", subcore_axis_name="s")
    out = pl.kernel(
        _sc_body,
        out_type=jax.ShapeDtypeStruct((BATCH,), jnp.float32),
        mesh=mesh,
        scratch_types=[
            pltpu.VMEM((_NCHUNK, _CHUNK), jnp.int32),
            pltpu.VMEM((_NCHUNK, _CHUNK), jnp.int32),
            pltpu.VMEM((_CHUNK, EMBED), jnp.float32),
            pltpu.VMEM((_CHUNK, EMBED), jnp.float32),
            pltpu.VMEM((16, 16), jnp.float32),
            pltpu.VMEM((_BPW,), jnp.float32),
            pltpu.VMEM((2, _L), jnp.float32),
            pltpu.SemaphoreType.DMA,
            pltpu.SemaphoreType.DMA,
        ],
    )(word_i, ctx_i, word_table, ctx_table, scal)
    return out.reshape(BATCH, 1)


# speedup vs baseline: 1.0709x; 1.0709x over previous
import functools

import jax
import jax.numpy as jnp
from jax import lax
from jax.experimental import pallas as pl
from jax.experimental.pallas import tpu as pltpu
from jax.experimental.pallas import tpu_sc as plsc

VOCAB = 100000
EMBED = 128
BATCH = 16384

_INFO = plsc.get_sparse_core_info()
_NC, _NS, _L = _INFO.num_cores, _INFO.num_subcores, _INFO.num_lanes
_NW = _NC * _NS
_BPW = BATCH // _NW
_CHUNK = 128
_NCHUNK = _BPW // _CHUNK
_GROUPS = _CHUNK // 16


def _sc_body(word_hbm, ctx_hbm, wt_hbm, ct_hbm, scal_hbm, out_hbm,
             idx_w, idx_c, wrows, crows, tbuf, res, scal_v, sem_w, sem_c):
    wid = lax.axis_index("s") * _NC + lax.axis_index("c")
    base = wid * _BPW

    for c in range(_NCHUNK):
        pltpu.sync_copy(word_hbm.at[pl.ds(base + c * _CHUNK, _CHUNK)],
                        idx_w.at[c])
        pltpu.sync_copy(ctx_hbm.at[pl.ds(base + c * _CHUNK, _CHUNK)],
                        idx_c.at[c])
    pltpu.sync_copy(scal_hbm, scal_v)
    wvec = scal_v[0, :]
    bvec = scal_v[1, :]
    iota16 = lax.iota(jnp.int32, _L)

    for c in range(_NCHUNK):
        cw = pltpu.async_copy(wt_hbm.at[idx_w.at[c]], wrows, sem_w)
        cc = pltpu.async_copy(ct_hbm.at[idx_c.at[c]], crows, sem_c)
        cw.wait()
        cc.wait()

        def group_body(g, _, c=c):
            row0 = g * 16
            for r in range(16):
                row = row0 + r
                p = wrows[row, pl.ds(0, 16)] * crows[row, pl.ds(0, 16)]
                for k in range(1, EMBED // 16):
                    p = p + (wrows[row, pl.ds(k * 16, 16)] *
                             crows[row, pl.ds(k * 16, 16)])
                tbuf[pl.ds(r * 16, 16)] = p
            tidx = iota16 * 16
            acc = plsc.load_gather(tbuf, [tidx])
            for j in range(1, 16):
                acc = acc + plsc.load_gather(tbuf, [tidx + j])
            z = acc * wvec + bvec
            res[pl.ds(c * _CHUNK + row0, 16)] = 1.0 / (1.0 + jnp.exp(-z))
            return 0

        lax.fori_loop(0, _GROUPS, group_body, 0)

    pltpu.sync_copy(res, out_hbm.at[pl.ds(base, _BPW)])


def kernel(word, context, word_table, ctx_table, dense_w, dense_b):
    word_i = word.reshape(-1).astype(jnp.int32)
    ctx_i = context.reshape(-1).astype(jnp.int32)
    scal = jnp.stack([
        jnp.broadcast_to(dense_w.reshape(()), (_L,)),
        jnp.broadcast_to(dense_b.reshape(()), (_L,)),
    ]).astype(jnp.float32)

    mesh = plsc.VectorSubcoreMesh(core_axis_name="c", subcore_axis_name="s")
    out = pl.kernel(
        _sc_body,
        out_type=jax.ShapeDtypeStruct((BATCH,), jnp.float32),
        mesh=mesh,
        compiler_params=pltpu.CompilerParams(needs_layout_passes=False),
        scratch_types=[
            pltpu.VMEM((_NCHUNK, _CHUNK), jnp.int32),
            pltpu.VMEM((_NCHUNK, _CHUNK), jnp.int32),
            pltpu.VMEM((_CHUNK, EMBED), jnp.float32),
            pltpu.VMEM((_CHUNK, EMBED), jnp.float32),
            pltpu.VMEM((256,), jnp.float32),
            pltpu.VMEM((_BPW,), jnp.float32),
            pltpu.VMEM((2, _L), jnp.float32),
            pltpu.SemaphoreType.DMA,
            pltpu.SemaphoreType.DMA,
        ],
    )(word_i, ctx_i, word_table, ctx_table, scal)
    return out.reshape(BATCH, 1)


# speedup vs baseline: 1.3711x; 1.2803x over previous
import jax
import jax.numpy as jnp
from jax import lax
from jax.experimental import pallas as pl
from jax.experimental.pallas import tpu as pltpu
from jax.experimental.pallas import tpu_sc as plsc

VOCAB = 100000
EMBED = 128
BATCH = 16384

_INFO = plsc.get_sparse_core_info()
_NC, _NS, _L = _INFO.num_cores, _INFO.num_subcores, _INFO.num_lanes
_NW = _NC * _NS
_BPW = BATCH // _NW
_CHUNK = 128
_NCHUNK = _BPW // _CHUNK
_GROUPS = _CHUNK // 16


def _sc_body(word_hbm, ctx_hbm, wt_hbm, ct_hbm, scal_hbm, out_hbm,
             idx_w, idx_c, wrows, crows, tbuf, res, scal_v,
             sem_iw, sem_ic, sem_w, sem_c):
    wid = lax.axis_index("s") * _NC + lax.axis_index("c")
    base = wid * _BPW

    ciw = pltpu.async_copy(word_hbm.at[pl.ds(base, _BPW)], idx_w, sem_iw)
    cic = pltpu.async_copy(ctx_hbm.at[pl.ds(base, _BPW)], idx_c, sem_ic)
    ciw.wait()
    cic.wait()

    pend = {}
    pend[0] = (
        pltpu.async_copy(wt_hbm.at[idx_w.at[pl.ds(0, _CHUNK)]],
                         wrows.at[0], sem_w),
        pltpu.async_copy(ct_hbm.at[idx_c.at[pl.ds(0, _CHUNK)]],
                         crows.at[0], sem_c),
    )
    pltpu.sync_copy(scal_hbm, scal_v)
    sv = scal_v[...]
    wvec = jnp.broadcast_to(sv[0], (_L,))
    bvec = jnp.broadcast_to(sv[1], (_L,))
    iota16 = lax.iota(jnp.int32, _L)

    for c in range(_NCHUNK):
        buf = c % 2
        if c + 1 < _NCHUNK:
            nb = (c + 1) % 2
            off = (c + 1) * _CHUNK
            pend[c + 1] = (
                pltpu.async_copy(wt_hbm.at[idx_w.at[pl.ds(off, _CHUNK)]],
                                 wrows.at[nb], sem_w),
                pltpu.async_copy(ct_hbm.at[idx_c.at[pl.ds(off, _CHUNK)]],
                                 crows.at[nb], sem_c),
            )
        cw, cc = pend.pop(c)
        cw.wait()
        cc.wait()

        def group_body(g, _, buf=buf, c=c):
            row0 = g * 16
            for r in range(16):
                row = row0 + r
                p = (wrows[buf, row, pl.ds(0, 16)] *
                     crows[buf, row, pl.ds(0, 16)])
                for k in range(1, EMBED // 16):
                    p = p + (wrows[buf, row, pl.ds(k * 16, 16)] *
                             crows[buf, row, pl.ds(k * 16, 16)])
                tbuf[pl.ds(r * 16, 16)] = p
            tidx = iota16 * 16
            acc = plsc.load_gather(tbuf, [tidx])
            for j in range(1, 16):
                acc = acc + plsc.load_gather(tbuf, [tidx + j])
            z = acc * wvec + bvec
            res[pl.ds(c * _CHUNK + row0, 16)] = 1.0 / (1.0 + jnp.exp(-z))
            return 0

        lax.fori_loop(0, _GROUPS, group_body, 0)

    pltpu.sync_copy(res, out_hbm.at[pl.ds(base, _BPW)])


def kernel(word, context, word_table, ctx_table, dense_w, dense_b):
    word_i = word.reshape(-1).astype(jnp.int32)
    ctx_i = context.reshape(-1).astype(jnp.int32)
    scal = jnp.concatenate([
        dense_w.reshape(-1).astype(jnp.float32),
        dense_b.reshape(-1).astype(jnp.float32),
        jnp.zeros((_L - 2,), jnp.float32),
    ])

    mesh = plsc.VectorSubcoreMesh(core_axis_name="c", subcore_axis_name="s")
    out = pl.kernel(
        _sc_body,
        out_type=jax.ShapeDtypeStruct((BATCH,), jnp.float32),
        mesh=mesh,
        compiler_params=pltpu.CompilerParams(needs_layout_passes=False),
        scratch_types=[
            pltpu.VMEM((_BPW,), jnp.int32),
            pltpu.VMEM((_BPW,), jnp.int32),
            pltpu.VMEM((2, _CHUNK, EMBED), jnp.float32),
            pltpu.VMEM((2, _CHUNK, EMBED), jnp.float32),
            pltpu.VMEM((256,), jnp.float32),
            pltpu.VMEM((_BPW,), jnp.float32),
            pltpu.VMEM((_L,), jnp.float32),
            pltpu.SemaphoreType.DMA,
            pltpu.SemaphoreType.DMA,
            pltpu.SemaphoreType.DMA,
            pltpu.SemaphoreType.DMA,
        ],
    )(word_i, ctx_i, word_table, ctx_table, scal)
    return out.reshape(BATCH, 1)


# speedup vs baseline: 1.3751x; 1.0029x over previous
import jax
import jax.numpy as jnp
from jax import lax
from jax.experimental import pallas as pl
from jax.experimental.pallas import tpu as pltpu
from jax.experimental.pallas import tpu_sc as plsc

VOCAB = 100000
EMBED = 128
BATCH = 16384

_INFO = plsc.get_sparse_core_info()
_NC, _NS, _L = _INFO.num_cores, _INFO.num_subcores, _INFO.num_lanes
_NW = _NC * _NS
_BPW = BATCH // _NW
_CHUNK = 128
_NCHUNK = _BPW // _CHUNK
_DEPTH = 3
_GROUPS = _CHUNK // 16


def _sc_body(word_hbm, ctx_hbm, wt_hbm, ct_hbm, dw_hbm, db_hbm, out_hbm,
             idx_w, idx_c, wrows, crows, tbuf, res, scal_v,
             sem_iw, sem_ic, sem_w, sem_c):
    wid = lax.axis_index("s") * _NC + lax.axis_index("c")
    base = wid * _BPW

    ciw = pltpu.async_copy(word_hbm.at[pl.ds(base, _BPW)], idx_w, sem_iw)
    cic = pltpu.async_copy(ctx_hbm.at[pl.ds(base, _BPW)], idx_c, sem_ic)
    ciw.wait()
    cic.wait()

    pend = {}

    def issue(c):
        off = c * _CHUNK
        slot = c % _DEPTH
        pend[c] = (
            pltpu.async_copy(wt_hbm.at[idx_w.at[pl.ds(off, _CHUNK)]],
                             wrows.at[slot], sem_w),
            pltpu.async_copy(ct_hbm.at[idx_c.at[pl.ds(off, _CHUNK)]],
                             crows.at[slot], sem_c),
        )

    for c in range(min(_DEPTH, _NCHUNK)):
        issue(c)
    pltpu.sync_copy(dw_hbm, scal_v.at[pl.ds(0, 1)])
    pltpu.sync_copy(db_hbm, scal_v.at[pl.ds(8, 1)])
    sv = scal_v[pl.ds(0, 16)]
    wvec = jnp.broadcast_to(sv[0], (_L,))
    bvec = jnp.broadcast_to(sv[8], (_L,))
    iota16 = lax.iota(jnp.int32, _L)

    for c in range(_NCHUNK):
        slot = c % _DEPTH
        cw, cc = pend.pop(c)
        cw.wait()
        cc.wait()

        def group_body(g, _, slot=slot, c=c):
            row0 = g * 16
            for r in range(16):
                row = row0 + r
                p = (wrows[slot, row, pl.ds(0, 16)] *
                     crows[slot, row, pl.ds(0, 16)])
                for k in range(1, EMBED // 16):
                    p = p + (wrows[slot, row, pl.ds(k * 16, 16)] *
                             crows[slot, row, pl.ds(k * 16, 16)])
                tbuf[pl.ds(r * 16, 16)] = p
            tidx = iota16 * 16
            acc = plsc.load_gather(tbuf, [tidx])
            for j in range(1, 16):
                acc = acc + plsc.load_gather(tbuf, [tidx + j])
            z = acc * wvec + bvec
            res[pl.ds(c * _CHUNK + row0, 16)] = 1.0 / (1.0 + jnp.exp(-z))
            return 0

        lax.fori_loop(0, _GROUPS, group_body, 0)
        if c + _DEPTH < _NCHUNK:
            issue(c + _DEPTH)

    pltpu.sync_copy(res, out_hbm.at[pl.ds(base, _BPW)])


def kernel(word, context, word_table, ctx_table, dense_w, dense_b):
    word_i = word.reshape(-1).astype(jnp.int32)
    ctx_i = context.reshape(-1).astype(jnp.int32)
    dw = dense_w.reshape(-1).astype(jnp.float32)
    db = dense_b.reshape(-1).astype(jnp.float32)

    mesh = plsc.VectorSubcoreMesh(core_axis_name="c", subcore_axis_name="s")
    out = pl.kernel(
        _sc_body,
        out_type=jax.ShapeDtypeStruct((BATCH,), jnp.float32),
        mesh=mesh,
        compiler_params=pltpu.CompilerParams(needs_layout_passes=False),
        scratch_types=[
            pltpu.VMEM((_BPW,), jnp.int32),
            pltpu.VMEM((_BPW,), jnp.int32),
            pltpu.VMEM((_DEPTH, _CHUNK, EMBED), jnp.float32),
            pltpu.VMEM((_DEPTH, _CHUNK, EMBED), jnp.float32),
            pltpu.VMEM((256,), jnp.float32),
            pltpu.VMEM((_BPW,), jnp.float32),
            pltpu.VMEM((32,), jnp.float32),
            pltpu.SemaphoreType.DMA,
            pltpu.SemaphoreType.DMA,
            pltpu.SemaphoreType.DMA,
            pltpu.SemaphoreType.DMA,
        ],
    )(word_i, ctx_i, word_table, ctx_table, dw, db)
    return out.reshape(BATCH, 1)
